# Initial kernel scaffold; baseline (speedup 1.0000x reference)
import jax
import jax.numpy as jnp
from jax.experimental import pallas as pl


def kernel(x, edge_index, edge_attr, batch, W1, b1, W2, b2, Wv, bv, Wp, bp):
    raise NotImplementedError("write your pallas kernel here")


# speedup vs baseline: 2.7912x; 2.7912x over previous
import functools

import jax
import jax.numpy as jnp
from jax import lax
from jax.experimental import pallas as pl
from jax.experimental.pallas import tpu as pltpu
from jax.experimental.pallas import tpu_sc as plsc

N = 10000
E = 320000
D = 128
B = 16
A = 1024

NC = 2
NS = 16
NW = NC * NS
CHUNK = 128
CPT = ((-(-E // (NW * CHUNK)) + 7) // 8) * 8
EP = NW * CPT * CHUNK
NP = 10240
RPT = NP // NS
NBLK = 1000
GRID = N // NBLK

_mesh = plsc.VectorSubcoreMesh(
    core_axis_name="c", subcore_axis_name="s", num_cores=NC, num_subcores=NS)
_sc_params = pltpu.CompilerParams(needs_layout_passes=False)


def _deg_body(dst2, w2, out, dst_slab, w_slab, zbuf, dacc):
    c = lax.axis_index("c")
    s = lax.axis_index("s")
    wid = c * NS + s

    @pl.when(s == 0)
    def _zero():
        for i in range(NP // 16):
            zbuf[pl.ds(i * 16, 16)] = jnp.zeros((16,), jnp.float32)
        pltpu.sync_copy(zbuf, dacc)

    plsc.subcore_barrier()
    pltpu.sync_copy(dst2.at[pl.ds(wid * CPT, CPT)], dst_slab)
    pltpu.sync_copy(w2.at[pl.ds(wid * CPT, CPT)], w_slab)

    def body(j, carry):
        pltpu.sync_copy(w_slab.at[j], dacc.at[dst_slab.at[j]], add=True)
        return carry

    lax.fori_loop(0, CPT, body, 0)
    plsc.subcore_barrier()
    pltpu.sync_copy(dacc.at[pl.ds(s * RPT, RPT)], out.at[c, pl.ds(s * RPT, RPT)])


_deg_call = pl.kernel(
    _deg_body,
    out_type=jax.ShapeDtypeStruct((NC, NP), jnp.float32),
    mesh=_mesh,
    scratch_types=[
        pltpu.VMEM((CPT, CHUNK), jnp.int32),
        pltpu.VMEM((CPT, CHUNK), jnp.float32),
        pltpu.VMEM((NP,), jnp.float32),
        pltpu.VMEM_SHARED((NP,), jnp.float32),
    ],
    compiler_params=_sc_params,
)


def _msg_body(xs, src2, dst2, w2, out, src_slab, dst_slab, w_slab, rows, acc):
    c = lax.axis_index("c")
    s = lax.axis_index("s")
    wid = c * NS + s
    iota16 = lax.iota(jnp.int32, 16)

    for r in range(CHUNK):
        for g in range(8):
            rows[r, pl.ds(g * 16, 16)] = jnp.zeros((16,), jnp.float32)
    for k in range(RPT // CHUNK):
        pltpu.sync_copy(rows, acc.at[pl.ds(s * RPT + k * CHUNK, CHUNK)])
    plsc.subcore_barrier()

    pltpu.sync_copy(src2.at[pl.ds(wid * CPT, CPT)], src_slab)
    pltpu.sync_copy(dst2.at[pl.ds(wid * CPT, CPT)], dst_slab)
    pltpu.sync_copy(w2.at[pl.ds(wid * CPT, CPT)], w_slab)

    def chunk_body(j, carry):
        pltpu.sync_copy(xs.at[src_slab.at[j]], rows)
        jv = jnp.full((16,), j, jnp.int32)
        wvs = tuple(
            plsc.load_gather(w_slab, [jv, g * 16 + iota16]) for g in range(8))

        def f_body(f, wv):
            colv = jnp.full((16,), f, jnp.int32)
            for g in range(8):
                rowv = g * 16 + iota16
                vals = plsc.load_gather(rows, [rowv, colv])
                plsc.store_scatter(rows, [rowv, colv], vals * wv[g])
            return wv

        lax.fori_loop(0, D, f_body, wvs)
        pltpu.sync_copy(rows, acc.at[dst_slab.at[j]], add=True)
        return carry

    lax.fori_loop(0, CPT, chunk_body, 0)
    plsc.subcore_barrier()
    for k in range(RPT // CHUNK):
        pltpu.sync_copy(acc.at[pl.ds(s * RPT + k * CHUNK, CHUNK)],
                        out.at[c, pl.ds(s * RPT + k * CHUNK, CHUNK)])


_msg_call = pl.kernel(
    _msg_body,
    out_type=jax.ShapeDtypeStruct((NC, NP, D), jnp.float32),
    mesh=_mesh,
    scratch_types=[
        pltpu.VMEM((CPT, CHUNK), jnp.int32),
        pltpu.VMEM((CPT, CHUNK), jnp.int32),
        pltpu.VMEM((CPT, CHUNK), jnp.float32),
        pltpu.VMEM((CHUNK, D), jnp.float32),
        pltpu.VMEM_SHARED((NP, D), jnp.float32),
    ],
    compiler_params=_sc_params,
)


def _tc1_body(deg_ref, x_ref, w1_ref, xs1_ref, dis_ref):
    deg = deg_ref[0] + deg_ref[1] + 1.0
    dis = lax.rsqrt(deg)
    xl = lax.dot_general(x_ref[...], w1_ref[...], (((1,), (1,)), ((), ())),
                         preferred_element_type=jnp.float32)
    xs1_ref[...] = dis * xl
    dis_ref[...] = dis


def _tc1_call(deg3, x, W1):
    return pl.pallas_call(
        _tc1_body,
        grid=(GRID,),
        in_specs=[
            pl.BlockSpec((NC, NBLK, 1), lambda i: (0, i, 0)),
            pl.BlockSpec((NBLK, D), lambda i: (i, 0)),
            pl.BlockSpec((D, D), lambda i: (0, 0)),
        ],
        out_specs=[
            pl.BlockSpec((NBLK, D), lambda i: (i, 0)),
            pl.BlockSpec((NBLK, 1), lambda i: (i, 0)),
        ],
        out_shape=[
            jax.ShapeDtypeStruct((N, D), jnp.float32),
            jax.ShapeDtypeStruct((N, 1), jnp.float32),
        ],
    )(deg3, x, W1)


def _tc2_body(acc_ref, xs1_ref, dis_ref, b1_ref, w2_ref, xs2_ref):
    dis = dis_ref[...]
    h = dis * (acc_ref[0] + acc_ref[1]) + dis * xs1_ref[...] + b1_ref[...]
    h = jnp.maximum(h, 0.0)
    xs2_ref[...] = dis * lax.dot_general(
        h, w2_ref[...], (((1,), (1,)), ((), ())),
        preferred_element_type=jnp.float32)


def _tc2_call(acc1, xs1, dis, b1r, W2):
    return pl.pallas_call(
        _tc2_body,
        grid=(GRID,),
        in_specs=[
            pl.BlockSpec((NC, NBLK, D), lambda i: (0, i, 0)),
            pl.BlockSpec((NBLK, D), lambda i: (i, 0)),
            pl.BlockSpec((NBLK, 1), lambda i: (i, 0)),
            pl.BlockSpec((1, D), lambda i: (0, 0)),
            pl.BlockSpec((D, D), lambda i: (0, 0)),
        ],
        out_specs=pl.BlockSpec((NBLK, D), lambda i: (i, 0)),
        out_shape=jax.ShapeDtypeStruct((N, D), jnp.float32),
    )(acc1, xs1, dis, b1r, W2)


def _tc3_body(acc_ref, xs2_ref, dis_ref, b2_ref, batch_ref, wv_ref, bv_ref,
              wp_ref, bp_ref, v_ref, p_ref, sums, cnts):
    i = pl.program_id(0)

    @pl.when(i == 0)
    def _init():
        sums[...] = jnp.zeros_like(sums)
        cnts[...] = jnp.zeros_like(cnts)

    dis = dis_ref[...]
    h = dis * (acc_ref[0] + acc_ref[1]) + dis * xs2_ref[...] + b2_ref[...]
    h = jnp.maximum(h, 0.0)
    onehot = (batch_ref[...] ==
              lax.broadcasted_iota(jnp.int32, (NBLK, B), 1)).astype(jnp.float32)
    sums[...] += lax.dot_general(onehot, h, (((0,), (0,)), ((), ())),
                                 preferred_element_type=jnp.float32)
    cnts[...] += lax.dot_general(onehot, jnp.ones((NBLK, D), jnp.float32),
                                 (((0,), (0,)), ((), ())),
                                 preferred_element_type=jnp.float32)

    @pl.when(i == pl.num_programs(0) - 1)
    def _final():
        g = sums[...] / jnp.maximum(cnts[...], 1.0)
        v = jnp.sum(g * wv_ref[...], axis=1, keepdims=True) + bv_ref[...]
        v_ref[...] = jnp.tanh(v)
        logits = lax.dot_general(g, wp_ref[...], (((1,), (1,)), ((), ())),
                                 preferred_element_type=jnp.float32) + bp_ref[...]
        m = jnp.max(logits, axis=1, keepdims=True)
        ex = jnp.exp(logits - m)
        p_ref[...] = ex / jnp.sum(ex, axis=1, keepdims=True)


def _tc3_call(acc2, xs2, dis, b2r, batch2, Wv, bvr, Wp, bpr):
    return pl.pallas_call(
        _tc3_body,
        grid=(GRID,),
        in_specs=[
            pl.BlockSpec((NC, NBLK, D), lambda i: (0, i, 0)),
            pl.BlockSpec((NBLK, D), lambda i: (i, 0)),
            pl.BlockSpec((NBLK, 1), lambda i: (i, 0)),
            pl.BlockSpec((1, D), lambda i: (0, 0)),
            pl.BlockSpec((NBLK, 1), lambda i: (i, 0)),
            pl.BlockSpec((1, D), lambda i: (0, 0)),
            pl.BlockSpec((1, 1), lambda i: (0, 0)),
            pl.BlockSpec((A, D), lambda i: (0, 0)),
            pl.BlockSpec((1, A), lambda i: (0, 0)),
        ],
        out_specs=[
            pl.BlockSpec((B, 1), lambda i: (0, 0)),
            pl.BlockSpec((B, A), lambda i: (0, 0)),
        ],
        out_shape=[
            jax.ShapeDtypeStruct((B, 1), jnp.float32),
            jax.ShapeDtypeStruct((B, A), jnp.float32),
        ],
        scratch_shapes=[
            pltpu.VMEM((B, D), jnp.float32),
            pltpu.VMEM((B, D), jnp.float32),
        ],
    )(acc2, xs2, dis, b2r, batch2, Wv, bvr, Wp, bpr)


def kernel(x, edge_index, edge_attr, batch, W1, b1, W2, b2, Wv, bv, Wp, bp):
    assert x.shape == (N, D) and edge_attr.shape == (E,)
    src = edge_index[0]
    dst = edge_index[1]
    pad = EP - E
    pad_idx = (jnp.arange(pad, dtype=jnp.int32) * 37) % N
    src2 = jnp.concatenate([src, pad_idx]).reshape(EP // CHUNK, CHUNK)
    dst2 = jnp.concatenate([dst, pad_idx]).reshape(EP // CHUNK, CHUNK)
    w2 = jnp.concatenate(
        [edge_attr, jnp.zeros((pad,), jnp.float32)]).reshape(EP // CHUNK, CHUNK)

    deg_parts = _deg_call(dst2, w2)
    xs1, dis = _tc1_call(deg_parts.reshape(NC, NP, 1), x, W1)
    acc1 = _msg_call(xs1, src2, dst2, w2)
    xs2 = _tc2_call(acc1, xs1, dis, b1.reshape(1, D), W2)
    acc2 = _msg_call(xs2, src2, dst2, w2)
    v, p = _tc3_call(acc2, xs2, dis, b2.reshape(1, D),
                     batch.reshape(N, 1), Wv, bv.reshape(1, 1),
                     Wp, bp.reshape(1, A))
    return (v, p)


# speedup vs baseline: 22.4482x; 8.0424x over previous
import functools

import jax
import jax.numpy as jnp
from jax import lax
from jax.experimental import pallas as pl
from jax.experimental.pallas import tpu as pltpu
from jax.experimental.pallas import tpu_sc as plsc

N = 10000
E = 320000
D = 128
B = 16
A = 1024

NC = 2
NS = 16
NW = NC * NS
CHUNK = 128
CPT = ((-(-E // (NW * CHUNK)) + 7) // 8) * 8
EP = NW * CPT * CHUNK
NP = 10240
RPT = NP // NS
NBLK = 1000
GRID = N // NBLK

_mesh = plsc.VectorSubcoreMesh(
    core_axis_name="c", subcore_axis_name="s", num_cores=NC, num_subcores=NS)
_sc_params = pltpu.CompilerParams(needs_layout_passes=False)


def _deg_body(dst2, w2, out, dst_slab, w_slab, zbuf, dacc):
    c = lax.axis_index("c")
    s = lax.axis_index("s")
    wid = c * NS + s

    @pl.when(s == 0)
    def _zero():
        for i in range(NP // 16):
            zbuf[pl.ds(i * 16, 16)] = jnp.zeros((16,), jnp.float32)
        pltpu.sync_copy(zbuf, dacc)

    plsc.subcore_barrier()
    pltpu.sync_copy(dst2.at[pl.ds(wid * CPT, CPT)], dst_slab)
    pltpu.sync_copy(w2.at[pl.ds(wid * CPT, CPT)], w_slab)

    def body(j, carry):
        pltpu.sync_copy(w_slab.at[j], dacc.at[dst_slab.at[j]], add=True)
        return carry

    lax.fori_loop(0, CPT, body, 0)
    plsc.subcore_barrier()
    pltpu.sync_copy(dacc.at[pl.ds(s * RPT, RPT)], out.at[c, pl.ds(s * RPT, RPT)])


_deg_call = pl.kernel(
    _deg_body,
    out_type=jax.ShapeDtypeStruct((NC, NP), jnp.float32),
    mesh=_mesh,
    scratch_types=[
        pltpu.VMEM((CPT, CHUNK), jnp.int32),
        pltpu.VMEM((CPT, CHUNK), jnp.float32),
        pltpu.VMEM((NP,), jnp.float32),
        pltpu.VMEM_SHARED((NP,), jnp.float32),
    ],
    compiler_params=_sc_params,
)


def _msg_body(xs, src2, dst2, w2, out, src_slab, dst_slab, w_slab, rows, acc):
    c = lax.axis_index("c")
    s = lax.axis_index("s")
    wid = c * NS + s
    iota16 = lax.iota(jnp.int32, 16)

    for r in range(CHUNK):
        for g in range(8):
            rows[r, pl.ds(g * 16, 16)] = jnp.zeros((16,), jnp.float32)
    for k in range(RPT // CHUNK):
        pltpu.sync_copy(rows, acc.at[pl.ds(s * RPT + k * CHUNK, CHUNK)])
    plsc.subcore_barrier()

    pltpu.sync_copy(src2.at[pl.ds(wid * CPT, CPT)], src_slab)
    pltpu.sync_copy(dst2.at[pl.ds(wid * CPT, CPT)], dst_slab)
    pltpu.sync_copy(w2.at[pl.ds(wid * CPT, CPT)], w_slab)

    def chunk_body(j, carry):
        pltpu.sync_copy(xs.at[src_slab.at[j]], rows)
        jv = jnp.full((16,), j, jnp.int32)
        wvs = tuple(
            plsc.load_gather(w_slab, [jv, g * 16 + iota16]) for g in range(8))

        @functools.partial(plsc.parallel_loop, 0, D, unroll=8)
        def _scale(f):
            colv = jnp.full((16,), f, jnp.int32)
            vals = [plsc.load_gather(rows, [g * 16 + iota16, colv])
                    for g in range(8)]
            for g in range(8):
                plsc.store_scatter(rows, [g * 16 + iota16, colv],
                                   vals[g] * wvs[g])

        pltpu.sync_copy(rows, acc.at[dst_slab.at[j]], add=True)
        return carry

    lax.fori_loop(0, CPT, chunk_body, 0)
    plsc.subcore_barrier()
    for k in range(RPT // CHUNK):
        pltpu.sync_copy(acc.at[pl.ds(s * RPT + k * CHUNK, CHUNK)],
                        out.at[c, pl.ds(s * RPT + k * CHUNK, CHUNK)])


_msg_call = pl.kernel(
    _msg_body,
    out_type=jax.ShapeDtypeStruct((NC, NP, D), jnp.float32),
    mesh=_mesh,
    scratch_types=[
        pltpu.VMEM((CPT, CHUNK), jnp.int32),
        pltpu.VMEM((CPT, CHUNK), jnp.int32),
        pltpu.VMEM((CPT, CHUNK), jnp.float32),
        pltpu.VMEM((CHUNK, D), jnp.float32),
        pltpu.VMEM_SHARED((NP, D), jnp.float32),
    ],
    compiler_params=_sc_params,
)


def _tc1_body(deg_ref, x_ref, w1_ref, xs1_ref, dis_ref):
    deg = deg_ref[0] + deg_ref[1] + 1.0
    dis = lax.rsqrt(deg)
    xl = lax.dot_general(x_ref[...], w1_ref[...], (((1,), (1,)), ((), ())),
                         preferred_element_type=jnp.float32)
    xs1_ref[...] = dis * xl
    dis_ref[...] = dis


def _tc1_call(deg3, x, W1):
    return pl.pallas_call(
        _tc1_body,
        grid=(GRID,),
        in_specs=[
            pl.BlockSpec((NC, NBLK, 1), lambda i: (0, i, 0)),
            pl.BlockSpec((NBLK, D), lambda i: (i, 0)),
            pl.BlockSpec((D, D), lambda i: (0, 0)),
        ],
        out_specs=[
            pl.BlockSpec((NBLK, D), lambda i: (i, 0)),
            pl.BlockSpec((NBLK, 1), lambda i: (i, 0)),
        ],
        out_shape=[
            jax.ShapeDtypeStruct((N, D), jnp.float32),
            jax.ShapeDtypeStruct((N, 1), jnp.float32),
        ],
    )(deg3, x, W1)


def _tc2_body(acc_ref, xs1_ref, dis_ref, b1_ref, w2_ref, xs2_ref):
    dis = dis_ref[...]
    h = dis * (acc_ref[0] + acc_ref[1]) + dis * xs1_ref[...] + b1_ref[...]
    h = jnp.maximum(h, 0.0)
    xs2_ref[...] = dis * lax.dot_general(
        h, w2_ref[...], (((1,), (1,)), ((), ())),
        preferred_element_type=jnp.float32)


def _tc2_call(acc1, xs1, dis, b1r, W2):
    return pl.pallas_call(
        _tc2_body,
        grid=(GRID,),
        in_specs=[
            pl.BlockSpec((NC, NBLK, D), lambda i: (0, i, 0)),
            pl.BlockSpec((NBLK, D), lambda i: (i, 0)),
            pl.BlockSpec((NBLK, 1), lambda i: (i, 0)),
            pl.BlockSpec((1, D), lambda i: (0, 0)),
            pl.BlockSpec((D, D), lambda i: (0, 0)),
        ],
        out_specs=pl.BlockSpec((NBLK, D), lambda i: (i, 0)),
        out_shape=jax.ShapeDtypeStruct((N, D), jnp.float32),
    )(acc1, xs1, dis, b1r, W2)


def _tc3_body(acc_ref, xs2_ref, dis_ref, b2_ref, batch_ref, wv_ref, bv_ref,
              wp_ref, bp_ref, v_ref, p_ref, sums, cnts):
    i = pl.program_id(0)

    @pl.when(i == 0)
    def _init():
        sums[...] = jnp.zeros_like(sums)
        cnts[...] = jnp.zeros_like(cnts)

    dis = dis_ref[...]
    h = dis * (acc_ref[0] + acc_ref[1]) + dis * xs2_ref[...] + b2_ref[...]
    h = jnp.maximum(h, 0.0)
    onehot = (batch_ref[...] ==
              lax.broadcasted_iota(jnp.int32, (NBLK, B), 1)).astype(jnp.float32)
    sums[...] += lax.dot_general(onehot, h, (((0,), (0,)), ((), ())),
                                 preferred_element_type=jnp.float32)
    cnts[...] += lax.dot_general(onehot, jnp.ones((NBLK, D), jnp.float32),
                                 (((0,), (0,)), ((), ())),
                                 preferred_element_type=jnp.float32)

    @pl.when(i == pl.num_programs(0) - 1)
    def _final():
        g = sums[...] / jnp.maximum(cnts[...], 1.0)
        v = jnp.sum(g * wv_ref[...], axis=1, keepdims=True) + bv_ref[...]
        v_ref[...] = jnp.tanh(v)
        logits = lax.dot_general(g, wp_ref[...], (((1,), (1,)), ((), ())),
                                 preferred_element_type=jnp.float32) + bp_ref[...]
        m = jnp.max(logits, axis=1, keepdims=True)
        ex = jnp.exp(logits - m)
        p_ref[...] = ex / jnp.sum(ex, axis=1, keepdims=True)


def _tc3_call(acc2, xs2, dis, b2r, batch2, Wv, bvr, Wp, bpr):
    return pl.pallas_call(
        _tc3_body,
        grid=(GRID,),
        in_specs=[
            pl.BlockSpec((NC, NBLK, D), lambda i: (0, i, 0)),
            pl.BlockSpec((NBLK, D), lambda i: (i, 0)),
            pl.BlockSpec((NBLK, 1), lambda i: (i, 0)),
            pl.BlockSpec((1, D), lambda i: (0, 0)),
            pl.BlockSpec((NBLK, 1), lambda i: (i, 0)),
            pl.BlockSpec((1, D), lambda i: (0, 0)),
            pl.BlockSpec((1, 1), lambda i: (0, 0)),
            pl.BlockSpec((A, D), lambda i: (0, 0)),
            pl.BlockSpec((1, A), lambda i: (0, 0)),
        ],
        out_specs=[
            pl.BlockSpec((B, 1), lambda i: (0, 0)),
            pl.BlockSpec((B, A), lambda i: (0, 0)),
        ],
        out_shape=[
            jax.ShapeDtypeStruct((B, 1), jnp.float32),
            jax.ShapeDtypeStruct((B, A), jnp.float32),
        ],
        scratch_shapes=[
            pltpu.VMEM((B, D), jnp.float32),
            pltpu.VMEM((B, D), jnp.float32),
        ],
    )(acc2, xs2, dis, b2r, batch2, Wv, bvr, Wp, bpr)


def kernel(x, edge_index, edge_attr, batch, W1, b1, W2, b2, Wv, bv, Wp, bp):
    assert x.shape == (N, D) and edge_attr.shape == (E,)
    src = edge_index[0]
    dst = edge_index[1]
    pad = EP - E
    pad_idx = (jnp.arange(pad, dtype=jnp.int32) * 37) % N
    src2 = jnp.concatenate([src, pad_idx]).reshape(EP // CHUNK, CHUNK)
    dst2 = jnp.concatenate([dst, pad_idx]).reshape(EP // CHUNK, CHUNK)
    w2 = jnp.concatenate(
        [edge_attr, jnp.zeros((pad,), jnp.float32)]).reshape(EP // CHUNK, CHUNK)

    deg_parts = _deg_call(dst2, w2)
    xs1, dis = _tc1_call(deg_parts.reshape(NC, NP, 1), x, W1)
    acc1 = _msg_call(xs1, src2, dst2, w2)
    xs2 = _tc2_call(acc1, xs1, dis, b1.reshape(1, D), W2)
    acc2 = _msg_call(xs2, src2, dst2, w2)
    v, p = _tc3_call(acc2, xs2, dis, b2.reshape(1, D),
                     batch.reshape(N, 1), Wv, bv.reshape(1, 1),
                     Wp, bp.reshape(1, A))
    return (v, p)
